# baseline (speedup 1.0000x reference)
import functools
import math

import jax
import jax.numpy as jnp
from jax import lax
from jax.experimental import pallas as pl
from jax.experimental.pallas import tpu as pltpu
from jax.experimental.pallas import tpu_sc as plsc

D_MODEL = 64
DPAD = 128
VPAD = 1000064
SCALE = math.sqrt(D_MODEL)

NC = 2
NS = 16
NW = NC * NS
G = 128
STEPS = 200
VCHUNKS = 7812
APAIRS = 123
VTAIL = VCHUNKS * G

_mesh = plsc.VectorSubcoreMesh(core_axis_name="c", subcore_axis_name="s")
_params = pltpu.CompilerParams(needs_layout_passes=False)


def _iotas(n):
    return [lax.iota(jnp.int32, 16) + 16 * g for g in range(n)]


@functools.partial(
    pl.kernel,
    mesh=_mesh,
    compiler_params=_params,
    out_type=jax.ShapeDtypeStruct((VPAD, DPAD), jnp.float32),
    scratch_types=[
        pltpu.VMEM((2, D_MODEL, G), jnp.float32),
        pltpu.VMEM((2, G, DPAD), jnp.float32),
        pltpu.SemaphoreType.DMA((2,)),
        pltpu.SemaphoreType.DMA((2,)),
    ],
)
def _relayout_scale(tt_hbm, ttail_hbm, out_hbm, inb, outb, gsem, ssem):
    wid = lax.axis_index("s") * NC + lax.axis_index("c")

    fiota = _iotas(8)

    for b in range(2):
        c = wid + NW * b

        @pl.when(c < VCHUNKS)
        def _():
            pltpu.make_async_copy(
                tt_hbm.at[:, pl.ds(c * G, G)], inb.at[b], gsem.at[b]
            ).start()

    def transpose_block(b, ngroups):
        for d in range(D_MODEL):
            dsplat = jnp.full((16,), d, jnp.int32)
            for g in range(ngroups):
                v = inb[b, d, pl.ds(16 * g, 16)] * SCALE
                plsc.store_scatter(outb.at[b], [fiota[g], dsplat], v)

    def pair(p, carry):
        for b in range(2):
            cc = 2 * p + b
            c = wid + NW * cc

            @pl.when(c < VCHUNKS)
            def _body():
                pltpu.make_async_copy(
                    tt_hbm.at[:, pl.ds(c * G, G)], inb.at[b], gsem.at[b]
                ).wait()

                @pl.when(cc >= 2)
                def _():
                    pltpu.make_async_copy(
                        outb.at[b], out_hbm.at[pl.ds(0, G)], ssem.at[b]
                    ).wait()

                transpose_block(b, 8)

                pltpu.make_async_copy(
                    outb.at[b], out_hbm.at[pl.ds(c * G, G)], ssem.at[b]
                ).start()

                c2 = c + 2 * NW

                @pl.when(c2 < VCHUNKS)
                def _():
                    pltpu.make_async_copy(
                        tt_hbm.at[:, pl.ds(c2 * G, G)], inb.at[b], gsem.at[b]
                    ).start()

        return carry

    lax.fori_loop(0, APAIRS, pair, 0)

    for b in range(2):
        pltpu.make_async_copy(
            outb.at[b], out_hbm.at[pl.ds(0, G)], ssem.at[b]
        ).wait()

    @pl.when(wid == 0)
    def _tail():
        pltpu.sync_copy(ttail_hbm, inb.at[0])
        transpose_block(0, 4)
        pltpu.sync_copy(
            outb.at[0, pl.ds(0, 64)], out_hbm.at[pl.ds(VTAIL, 64)]
        )


@functools.partial(
    pl.kernel,
    mesh=_mesh,
    compiler_params=_params,
    out_type=jax.ShapeDtypeStruct((STEPS, D_MODEL, 4096), jnp.float32),
    scratch_types=[
        pltpu.VMEM((STEPS, G), jnp.int32),
        pltpu.VMEM((2, G, DPAD), jnp.float32),
        pltpu.VMEM((2, D_MODEL, G), jnp.float32),
        pltpu.SemaphoreType.DMA((2,)),
        pltpu.SemaphoreType.DMA((2,)),
    ],
)
def _emb_lookup(table_hbm, idx_hbm, out_hbm, idx_v, inb, outb, gsem, ssem):
    wid = lax.axis_index("s") * NC + lax.axis_index("c")
    bbase = wid * G
    pltpu.sync_copy(idx_hbm.at[:, pl.ds(bbase, G)], idx_v)

    giota = _iotas(4)

    for b in range(2):
        pltpu.make_async_copy(
            table_hbm.at[idx_v.at[b]], inb.at[b], gsem.at[b]
        ).start()

    def pair(p, carry):
        for b in range(2):
            i = 2 * p + b
            pltpu.make_async_copy(
                table_hbm.at[idx_v.at[i]], inb.at[b], gsem.at[b]
            ).wait()

            @pl.when(i >= 2)
            def _():
                pltpu.make_async_copy(
                    outb.at[b], out_hbm.at[0, :, pl.ds(bbase, G)], ssem.at[b]
                ).wait()

            for j in range(G):
                jsplat = jnp.full((16,), j, jnp.int32)
                for g in range(4):
                    v = inb[b, j, pl.ds(16 * g, 16)]
                    plsc.store_scatter(outb.at[b], [giota[g], jsplat], v)

            pltpu.make_async_copy(
                outb.at[b], out_hbm.at[i, :, pl.ds(bbase, G)], ssem.at[b]
            ).start()

            @pl.when(i + 2 < STEPS)
            def _():
                pltpu.make_async_copy(
                    table_hbm.at[idx_v.at[i + 2]], inb.at[b], gsem.at[b]
                ).start()

        return carry

    lax.fori_loop(0, STEPS // 2, pair, 0)

    for b in range(2):
        pltpu.make_async_copy(
            outb.at[b], out_hbm.at[0, :, pl.ds(bbase, G)], ssem.at[b]
        ).wait()


def kernel(x, table):
    tt = jnp.transpose(table)
    ttail = jnp.pad(tt[:, VTAIL:], ((0, 0), (0, 64)))
    tsc = _relayout_scale(tt, ttail)
    xt = jnp.transpose(x)
    out = _emb_lookup(tsc, xt)
    return jnp.transpose(out, (2, 0, 1))


# speedup vs baseline: 1.0332x; 1.0332x over previous
import functools
import math

import jax
import jax.numpy as jnp
from jax import lax
from jax.experimental import pallas as pl
from jax.experimental.pallas import tpu as pltpu
from jax.experimental.pallas import tpu_sc as plsc

D_MODEL = 64
DPAD = 128
VPAD = 1000064
SCALE = math.sqrt(D_MODEL)

NC = 2
NS = 16
NW = NC * NS
G = 128
STEPS = 200
VCHUNKS = 7812
APAIRS = 123
VTAIL = VCHUNKS * G

_mesh = plsc.VectorSubcoreMesh(core_axis_name="c", subcore_axis_name="s")
_params = pltpu.CompilerParams(needs_layout_passes=False)


def _iotas(n):
    return [lax.iota(jnp.int32, 16) + 16 * g for g in range(n)]


@functools.partial(
    pl.kernel,
    mesh=_mesh,
    compiler_params=_params,
    out_type=jax.ShapeDtypeStruct((VPAD, DPAD), jnp.float32),
    scratch_types=[
        pltpu.VMEM((2, D_MODEL, G), jnp.float32),
        pltpu.VMEM((2, G, DPAD + 1), jnp.float32),
        pltpu.SemaphoreType.DMA((2,)),
        pltpu.SemaphoreType.DMA((2,)),
    ],
)
def _relayout_scale(tt_hbm, ttail_hbm, out_hbm, inb, outb, gsem, ssem):
    wid = lax.axis_index("s") * NC + lax.axis_index("c")

    fiota = _iotas(8)

    for b in range(2):
        c = wid + NW * b

        @pl.when(c < VCHUNKS)
        def _():
            pltpu.make_async_copy(
                tt_hbm.at[:, pl.ds(c * G, G)], inb.at[b], gsem.at[b]
            ).start()

    def transpose_block(b, ngroups):
        def col(d, cr):
            dsplat = jnp.full((16,), d, jnp.int32)
            for g in range(ngroups):
                v = inb[b, d, pl.ds(16 * g, 16)] * SCALE
                plsc.store_scatter(outb.at[b], [fiota[g], dsplat], v)
            return cr

        lax.fori_loop(0, D_MODEL, col, 0, unroll=2)

    def pair(p, carry):
        for b in range(2):
            cc = 2 * p + b
            c = wid + NW * cc

            @pl.when(c < VCHUNKS)
            def _body():
                pltpu.make_async_copy(
                    tt_hbm.at[:, pl.ds(c * G, G)], inb.at[b], gsem.at[b]
                ).wait()

                @pl.when(cc >= 2)
                def _():
                    pltpu.make_async_copy(
                        outb.at[b, :, pl.ds(0, DPAD)],
                        out_hbm.at[pl.ds(0, G)],
                        ssem.at[b],
                    ).wait()

                transpose_block(b, 8)

                pltpu.make_async_copy(
                    outb.at[b, :, pl.ds(0, DPAD)],
                    out_hbm.at[pl.ds(c * G, G)],
                    ssem.at[b],
                ).start()

                c2 = c + 2 * NW

                @pl.when(c2 < VCHUNKS)
                def _():
                    pltpu.make_async_copy(
                        tt_hbm.at[:, pl.ds(c2 * G, G)], inb.at[b], gsem.at[b]
                    ).start()

        return carry

    lax.fori_loop(0, APAIRS, pair, 0)

    for b in range(2):
        pltpu.make_async_copy(
            outb.at[b, :, pl.ds(0, DPAD)], out_hbm.at[pl.ds(0, G)], ssem.at[b]
        ).wait()

    @pl.when(wid == 0)
    def _tail():
        pltpu.sync_copy(ttail_hbm, inb.at[0])
        transpose_block(0, 4)
        pltpu.sync_copy(
            outb.at[0, pl.ds(0, 64), pl.ds(0, DPAD)],
            out_hbm.at[pl.ds(VTAIL, 64)],
        )


@functools.partial(
    pl.kernel,
    mesh=_mesh,
    compiler_params=_params,
    out_type=jax.ShapeDtypeStruct((STEPS, D_MODEL, 4096), jnp.float32),
    scratch_types=[
        pltpu.VMEM((STEPS, G), jnp.int32),
        pltpu.VMEM((2, G, DPAD), jnp.float32),
        pltpu.VMEM((2, D_MODEL, G + 1), jnp.float32),
        pltpu.SemaphoreType.DMA((2,)),
        pltpu.SemaphoreType.DMA((2,)),
    ],
)
def _emb_lookup(table_hbm, idx_hbm, out_hbm, idx_v, inb, outb, gsem, ssem):
    wid = lax.axis_index("s") * NC + lax.axis_index("c")
    bbase = wid * G
    pltpu.sync_copy(idx_hbm.at[:, pl.ds(bbase, G)], idx_v)

    giota = _iotas(4)

    for b in range(2):
        pltpu.make_async_copy(
            table_hbm.at[idx_v.at[b]], inb.at[b], gsem.at[b]
        ).start()

    def pair(p, carry):
        for b in range(2):
            i = 2 * p + b
            pltpu.make_async_copy(
                table_hbm.at[idx_v.at[i]], inb.at[b], gsem.at[b]
            ).wait()

            @pl.when(i >= 2)
            def _():
                pltpu.make_async_copy(
                    outb.at[b, :, pl.ds(0, G)],
                    out_hbm.at[0, :, pl.ds(bbase, G)],
                    ssem.at[b],
                ).wait()

            def tok(j, cr):
                jsplat = jnp.full((16,), j, jnp.int32)
                for g in range(4):
                    v = inb[b, j, pl.ds(16 * g, 16)]
                    plsc.store_scatter(outb.at[b], [giota[g], jsplat], v)
                return cr

            lax.fori_loop(0, G, tok, 0, unroll=2)

            pltpu.make_async_copy(
                outb.at[b, :, pl.ds(0, G)],
                out_hbm.at[i, :, pl.ds(bbase, G)],
                ssem.at[b],
            ).start()

            @pl.when(i + 2 < STEPS)
            def _():
                pltpu.make_async_copy(
                    table_hbm.at[idx_v.at[i + 2]], inb.at[b], gsem.at[b]
                ).start()

        return carry

    lax.fori_loop(0, STEPS // 2, pair, 0)

    for b in range(2):
        pltpu.make_async_copy(
            outb.at[b, :, pl.ds(0, G)],
            out_hbm.at[0, :, pl.ds(bbase, G)],
            ssem.at[b],
        ).wait()


def kernel(x, table):
    tt = jnp.transpose(table)
    ttail = jnp.pad(tt[:, VTAIL:], ((0, 0), (0, 64)))
    tsc = _relayout_scale(tt, ttail)
    xt = jnp.transpose(x)
    out = _emb_lookup(tsc, xt)
    return jnp.transpose(out, (2, 0, 1))


# speedup vs baseline: 1.5445x; 1.4949x over previous
import functools
import math

import jax
import jax.numpy as jnp
from jax import lax
from jax.experimental import pallas as pl
from jax.experimental.pallas import tpu as pltpu
from jax.experimental.pallas import tpu_sc as plsc

D_MODEL = 64
DPAD = 128
VPAD = 1000064
SCALE = math.sqrt(D_MODEL)

NC = 2
NS = 16
NW = NC * NS
G = 128
STEPS = 200
VCHUNKS = 7812
APAIRS = 123
VTAIL = VCHUNKS * G

_mesh = plsc.VectorSubcoreMesh(core_axis_name="c", subcore_axis_name="s")
_params = pltpu.CompilerParams(needs_layout_passes=False)


def _iotas(n):
    return [lax.iota(jnp.int32, 16) + 16 * g for g in range(n)]


@functools.partial(
    pl.kernel,
    mesh=_mesh,
    compiler_params=_params,
    out_type=jax.ShapeDtypeStruct((VPAD, DPAD), jnp.float32),
    scratch_types=[
        pltpu.VMEM((2, D_MODEL, G), jnp.float32),
        pltpu.VMEM((2, G, DPAD + 1), jnp.float32),
        pltpu.SemaphoreType.DMA((2,)),
        pltpu.SemaphoreType.DMA((2,)),
    ],
)
def _relayout_scale(tt_hbm, ttail_hbm, out_hbm, inb, outb, gsem, ssem):
    wid = lax.axis_index("s") * NC + lax.axis_index("c")

    fiota = _iotas(8)

    for b in range(2):
        c = wid + NW * b

        @pl.when(c < VCHUNKS)
        def _():
            pltpu.make_async_copy(
                tt_hbm.at[:, pl.ds(c * G, G)], inb.at[b], gsem.at[b]
            ).start()

    def transpose_block(b, ngroups):
        @plsc.parallel_loop(0, D_MODEL, unroll=8)
        def col(d):
            dsplat = jnp.full((16,), d, jnp.int32)
            for g in range(ngroups):
                v = inb[b, d, pl.ds(16 * g, 16)] * SCALE
                plsc.store_scatter(outb.at[b], [fiota[g], dsplat], v)

    def pair(p, carry):
        for b in range(2):
            cc = 2 * p + b
            c = wid + NW * cc

            @pl.when(c < VCHUNKS)
            def _body():
                pltpu.make_async_copy(
                    tt_hbm.at[:, pl.ds(c * G, G)], inb.at[b], gsem.at[b]
                ).wait()

                @pl.when(cc >= 2)
                def _():
                    pltpu.make_async_copy(
                        outb.at[b, :, pl.ds(0, DPAD)],
                        out_hbm.at[pl.ds(0, G)],
                        ssem.at[b],
                    ).wait()

                transpose_block(b, 8)

                pltpu.make_async_copy(
                    outb.at[b, :, pl.ds(0, DPAD)],
                    out_hbm.at[pl.ds(c * G, G)],
                    ssem.at[b],
                ).start()

                c2 = c + 2 * NW

                @pl.when(c2 < VCHUNKS)
                def _():
                    pltpu.make_async_copy(
                        tt_hbm.at[:, pl.ds(c2 * G, G)], inb.at[b], gsem.at[b]
                    ).start()

        return carry

    lax.fori_loop(0, APAIRS, pair, 0)

    for b in range(2):
        pltpu.make_async_copy(
            outb.at[b, :, pl.ds(0, DPAD)], out_hbm.at[pl.ds(0, G)], ssem.at[b]
        ).wait()

    @pl.when(wid == 0)
    def _tail():
        pltpu.sync_copy(ttail_hbm, inb.at[0])
        transpose_block(0, 4)
        pltpu.sync_copy(
            outb.at[0, pl.ds(0, 64), pl.ds(0, DPAD)],
            out_hbm.at[pl.ds(VTAIL, 64)],
        )


@functools.partial(
    pl.kernel,
    mesh=_mesh,
    compiler_params=_params,
    out_type=jax.ShapeDtypeStruct((STEPS, D_MODEL, 4096), jnp.float32),
    scratch_types=[
        pltpu.VMEM((STEPS, G), jnp.int32),
        pltpu.VMEM((2, G, DPAD), jnp.float32),
        pltpu.VMEM((2, D_MODEL, G + 1), jnp.float32),
        pltpu.SemaphoreType.DMA((2,)),
        pltpu.SemaphoreType.DMA((2,)),
    ],
)
def _emb_lookup(table_hbm, idx_hbm, out_hbm, idx_v, inb, outb, gsem, ssem):
    wid = lax.axis_index("s") * NC + lax.axis_index("c")
    bbase = wid * G
    pltpu.sync_copy(idx_hbm.at[:, pl.ds(bbase, G)], idx_v)

    giota = _iotas(4)

    for b in range(2):
        pltpu.make_async_copy(
            table_hbm.at[idx_v.at[b]], inb.at[b], gsem.at[b]
        ).start()

    def pair(p, carry):
        for b in range(2):
            i = 2 * p + b
            pltpu.make_async_copy(
                table_hbm.at[idx_v.at[i]], inb.at[b], gsem.at[b]
            ).wait()

            @pl.when(i >= 2)
            def _():
                pltpu.make_async_copy(
                    outb.at[b, :, pl.ds(0, G)],
                    out_hbm.at[0, :, pl.ds(bbase, G)],
                    ssem.at[b],
                ).wait()

            @plsc.parallel_loop(0, G, unroll=8)
            def tok(j):
                jsplat = jnp.full((16,), j, jnp.int32)
                for g in range(4):
                    v = inb[b, j, pl.ds(16 * g, 16)]
                    plsc.store_scatter(outb.at[b], [giota[g], jsplat], v)

            pltpu.make_async_copy(
                outb.at[b, :, pl.ds(0, G)],
                out_hbm.at[i, :, pl.ds(bbase, G)],
                ssem.at[b],
            ).start()

            @pl.when(i + 2 < STEPS)
            def _():
                pltpu.make_async_copy(
                    table_hbm.at[idx_v.at[i + 2]], inb.at[b], gsem.at[b]
                ).start()

        return carry

    lax.fori_loop(0, STEPS // 2, pair, 0)

    for b in range(2):
        pltpu.make_async_copy(
            outb.at[b, :, pl.ds(0, G)],
            out_hbm.at[0, :, pl.ds(bbase, G)],
            ssem.at[b],
        ).wait()


def kernel(x, table):
    tt = jnp.transpose(table)
    ttail = jnp.pad(tt[:, VTAIL:], ((0, 0), (0, 64)))
    tsc = _relayout_scale(tt, ttail)
    xt = jnp.transpose(x)
    out = _emb_lookup(tsc, xt)
    return jnp.transpose(out, (2, 0, 1))


# speedup vs baseline: 1.9575x; 1.2674x over previous
import functools
import math

import jax
import jax.numpy as jnp
from jax import lax
from jax.experimental import pallas as pl
from jax.experimental.pallas import tpu as pltpu
from jax.experimental.pallas import tpu_sc as plsc

D_MODEL = 64
DPAD = 128
SCALE = math.sqrt(D_MODEL)

NC = 2
NS = 16
NW = NC * NS
G = 128
STEPS = 200
NBUF = 2

_mesh = plsc.VectorSubcoreMesh(core_axis_name="c", subcore_axis_name="s")


@functools.partial(
    pl.kernel,
    mesh=_mesh,
    out_type=jax.ShapeDtypeStruct((4096, STEPS, D_MODEL), jnp.float32),
    scratch_types=[
        pltpu.VMEM((STEPS, G), jnp.int32),
        pltpu.VMEM((NBUF, G, DPAD), jnp.float32),
        pltpu.VMEM((NBUF, G, D_MODEL), jnp.float32),
        pltpu.SemaphoreType.DMA((NBUF,)),
        pltpu.SemaphoreType.DMA((NBUF,)),
    ],
)
def _emb_lookup(table_hbm, idx_hbm, out_hbm, idx_v, inb, outb, gsem, ssem):
    wid = lax.axis_index("s") * NC + lax.axis_index("c")
    bbase = wid * G
    pltpu.sync_copy(idx_hbm.at[:, pl.ds(bbase, G)], idx_v)

    for b in range(NBUF):
        pltpu.make_async_copy(
            table_hbm.at[idx_v.at[b]], inb.at[b], gsem.at[b]
        ).start()

    def group(g, carry):
        for b in range(NBUF):
            i = g * NBUF + b
            pltpu.make_async_copy(
                table_hbm.at[idx_v.at[i]], inb.at[b], gsem.at[b]
            ).wait()

            @pl.when(g > 0)
            def _wait_store():
                pltpu.make_async_copy(
                    outb.at[b], out_hbm.at[pl.ds(bbase, G), 0], ssem.at[b]
                ).wait()

            def row(r, c):
                for j in range(D_MODEL // 16):
                    sl = pl.ds(j * 16, 16)
                    outb[b, r, sl] = inb[b, r, sl] * SCALE
                return c

            lax.fori_loop(0, G, row, 0, unroll=4)

            pltpu.make_async_copy(
                outb.at[b], out_hbm.at[pl.ds(bbase, G), i], ssem.at[b]
            ).start()

            @pl.when(g < STEPS // NBUF - 1)
            def _next_gather():
                pltpu.make_async_copy(
                    table_hbm.at[idx_v.at[i + NBUF]], inb.at[b], gsem.at[b]
                ).start()

        return carry

    lax.fori_loop(0, STEPS // NBUF, group, 0)

    for b in range(NBUF):
        pltpu.make_async_copy(
            outb.at[b], out_hbm.at[pl.ds(bbase, G), 0], ssem.at[b]
        ).wait()


def kernel(x, table):
    tpad = jnp.pad(table, ((0, 0), (0, DPAD - D_MODEL)))
    xt = jnp.transpose(x)
    out = _emb_lookup(tpad, xt)
    return out
